# baseline (speedup 1.0000x reference)
import dataclasses
import functools

import jax
import jax.numpy as jnp
from jax import lax
from jax.experimental import pallas as pl
from jax.experimental.pallas import tpu as pltpu

N = 4096
B = 4096
K = 64
RBK = 256
CW = 128


def _tc_body(f_ref, x_ref, o_ref):
    xv = x_ref[...]
    x1 = jnp.floor(xv).astype(jnp.int32)
    dx = xv - x1.astype(jnp.float32)
    x2 = x1 + 1
    c1, l1 = lax.shift_right_logical(x1, 7), lax.bitwise_and(x1, 127)
    c2, l2 = lax.shift_right_logical(x2, 7), lax.bitwise_and(x2, 127)
    y1 = jnp.zeros((RBK, K), jnp.float32)
    y2 = jnp.zeros((RBK, K), jnp.float32)
    for v in range(B // CW):
        fb = f_ref[:, v * CW:(v + 1) * CW]
        y1 = jnp.where(c1 == v, jnp.take_along_axis(fb, l1, axis=1), y1)
        y2 = jnp.where(c2 == v, jnp.take_along_axis(fb, l2, axis=1), y2)
    o_ref[...] = y1 * (1.0 - dx) + y2 * dx


@jax.jit
def kernel(f, x):
    grid = (N // RBK,)
    return pl.pallas_call(
        _tc_body,
        grid=grid,
        in_specs=[
            pl.BlockSpec((RBK, B), lambda i: (i, 0)),
            pl.BlockSpec((RBK, K), lambda i: (i, 0)),
        ],
        out_specs=pl.BlockSpec((RBK, K), lambda i: (i, 0)),
        out_shape=jax.ShapeDtypeStruct((N, K), jnp.float32),
    )(f, x)


# speedup vs baseline: 2.0590x; 2.0590x over previous
import dataclasses

import jax
import jax.numpy as jnp
from jax import lax
from jax.experimental import pallas as pl
from jax.experimental.pallas import tpu as pltpu
from jax.experimental.pallas import tpu_sc as plsc

N = 4096
B = 4096
K = 64
S = 2048


NC, NS, L = 2, 16, 16
NW = NC * NS
RPW = S // NW
RB = 4
NBLK = RPW // RB
NBUF = 4


def _sc_kernel(f_hbm, x_hbm, o_hbm, b0, b1, b2, b3, x_v, o_v, s0, s1, s2, s3):
    wid = lax.axis_index("s") * NC + lax.axis_index("c")
    row0 = wid * RPW
    bufs = [b0, b1, b2, b3]
    sems = [s0, s1, s2, s3]

    def start(blk, buf, sem):
        pltpu.async_copy(f_hbm.at[pl.ds(row0 + blk * RB, RB)], buf, sem)

    def wait(buf, sem):
        pltpu.make_async_copy(f_hbm.at[pl.ds(row0, RB)], buf, sem).wait()

    def compute(blk, buf):
        @pl.loop(0, RB)
        def _row(r):
            rloc = blk * RB + r
            rv = jnp.full((L,), r, jnp.int32)
            for c in range(K // L):
                xv = x_v[rloc, pl.ds(c * L, L)]
                x1 = xv.astype(jnp.int32)
                dx = xv - x1.astype(jnp.float32)
                y1 = plsc.load_gather(buf, [rv, x1])
                y2 = plsc.load_gather(buf, [rv, x1 + 1])
                o_v[rloc, pl.ds(c * L, L)] = y1 * (1.0 - dx) + y2 * dx

    for k in range(NBUF):
        start(k, bufs[k], sems[k])
    pltpu.sync_copy(x_hbm.at[pl.ds(row0, RPW)], x_v)

    @pl.loop(0, NBLK, step=NBUF)
    def _blk(b):
        for k in range(NBUF):
            wait(bufs[k], sems[k])
            compute(b + k, bufs[k])

            @pl.when(b + k + NBUF < NBLK)
            def _():
                start(b + k + NBUF, bufs[k], sems[k])

    pltpu.sync_copy(o_v, o_hbm.at[pl.ds(row0, RPW)])


def _compiler_params():
    cp = pltpu.CompilerParams()
    if "needs_layout_passes" in pltpu.CompilerParams.__dataclass_fields__:
        cp = dataclasses.replace(cp, needs_layout_passes=False)
    return cp


def _sc_half(f, x):
    mesh = plsc.VectorSubcoreMesh(core_axis_name="c", subcore_axis_name="s")
    run = pl.kernel(
        _sc_kernel,
        out_type=jax.ShapeDtypeStruct((S, K), jnp.float32),
        mesh=mesh,
        scratch_types=[
            pltpu.VMEM((RB, B), jnp.float32),
            pltpu.VMEM((RB, B), jnp.float32),
            pltpu.VMEM((RB, B), jnp.float32),
            pltpu.VMEM((RB, B), jnp.float32),
            pltpu.VMEM((RPW, K), jnp.float32),
            pltpu.VMEM((RPW, K), jnp.float32),
            pltpu.SemaphoreType.DMA,
            pltpu.SemaphoreType.DMA,
            pltpu.SemaphoreType.DMA,
            pltpu.SemaphoreType.DMA,
        ],
        compiler_params=_compiler_params(),
    )
    return run(f, x)


RBK = 256
CW = 128


def _tc_body(f_ref, x_ref, o_ref):
    xv = x_ref[...]
    x1 = jnp.floor(xv).astype(jnp.int32)
    dx = xv - x1.astype(jnp.float32)
    c1, l1 = lax.shift_right_logical(x1, 7), lax.bitwise_and(x1, 127)
    y1 = jnp.zeros((RBK, K), jnp.float32)
    for v in range(B // CW):
        fb = f_ref[:, v * CW:(v + 1) * CW]
        y1 = jnp.where(c1 == v, jnp.take_along_axis(fb, l1, axis=1), y1)
    o_ref[...] = y1

    @pl.when(jnp.any(dx != 0.0))
    def _slow():
        x2 = x1 + 1
        c2, l2 = lax.shift_right_logical(x2, 7), lax.bitwise_and(x2, 127)
        y2 = jnp.zeros((RBK, K), jnp.float32)
        for v in range(B // CW):
            fb = f_ref[:, v * CW:(v + 1) * CW]
            y2 = jnp.where(c2 == v, jnp.take_along_axis(fb, l2, axis=1), y2)
        o_ref[...] = y1 * (1.0 - dx) + y2 * dx


def _tc_half(f, x):
    grid = ((N - S) // RBK,)
    return pl.pallas_call(
        _tc_body,
        grid=grid,
        in_specs=[
            pl.BlockSpec((RBK, B), lambda i: (S // RBK + i, 0)),
            pl.BlockSpec((RBK, K), lambda i: (S // RBK + i, 0)),
        ],
        out_specs=pl.BlockSpec((RBK, K), lambda i: (i, 0)),
        out_shape=jax.ShapeDtypeStruct((N - S, K), jnp.float32),
    )(f, x)


@jax.jit
def kernel(f, x):
    return jnp.concatenate([_sc_half(f, x), _tc_half(f, x)], axis=0)


# speedup vs baseline: 2.2385x; 1.0872x over previous
import dataclasses

import jax
import jax.numpy as jnp
from jax import lax
from jax.experimental import pallas as pl
from jax.experimental.pallas import tpu as pltpu
from jax.experimental.pallas import tpu_sc as plsc

N = 4096
B = 4096
K = 64
S = 3072


NC, NS, L = 2, 16, 16
NW = NC * NS
RPW = S // NW
RB = 4
NBLK = RPW // RB
NBUF = 4


def _sc_kernel(f_hbm, x_hbm, o_hbm, b0, b1, b2, b3, x_v, o_v, s0, s1, s2, s3):
    wid = lax.axis_index("s") * NC + lax.axis_index("c")
    row0 = wid * RPW
    bufs = [b0, b1, b2, b3]
    sems = [s0, s1, s2, s3]

    def start(blk, buf, sem):
        pltpu.async_copy(f_hbm.at[pl.ds(row0 + blk * RB, RB)], buf, sem)

    def wait(buf, sem):
        pltpu.make_async_copy(f_hbm.at[pl.ds(row0, RB)], buf, sem).wait()

    def compute(blk, buf):
        @pl.loop(0, RB)
        def _row(r):
            rloc = blk * RB + r
            rv = jnp.full((L,), r, jnp.int32)
            for c in range(K // L):
                xv = x_v[rloc, pl.ds(c * L, L)]
                x1 = xv.astype(jnp.int32)
                dx = xv - x1.astype(jnp.float32)
                y1 = plsc.load_gather(buf, [rv, x1])
                y2 = plsc.load_gather(buf, [rv, x1 + 1])
                o_v[rloc, pl.ds(c * L, L)] = y1 * (1.0 - dx) + y2 * dx

    for k in range(NBUF):
        start(k, bufs[k], sems[k])
    pltpu.sync_copy(x_hbm.at[pl.ds(row0, RPW)], x_v)

    @pl.loop(0, NBLK, step=NBUF)
    def _blk(b):
        for k in range(NBUF):
            wait(bufs[k], sems[k])
            compute(b + k, bufs[k])

            @pl.when(b + k + NBUF < NBLK)
            def _():
                start(b + k + NBUF, bufs[k], sems[k])

    pltpu.sync_copy(o_v, o_hbm.at[pl.ds(row0, RPW)])


def _compiler_params():
    cp = pltpu.CompilerParams()
    if "needs_layout_passes" in pltpu.CompilerParams.__dataclass_fields__:
        cp = dataclasses.replace(cp, needs_layout_passes=False)
    return cp


def _sc_half(f, x):
    mesh = plsc.VectorSubcoreMesh(core_axis_name="c", subcore_axis_name="s")
    run = pl.kernel(
        _sc_kernel,
        out_type=jax.ShapeDtypeStruct((S, K), jnp.float32),
        mesh=mesh,
        scratch_types=[
            pltpu.VMEM((RB, B), jnp.float32),
            pltpu.VMEM((RB, B), jnp.float32),
            pltpu.VMEM((RB, B), jnp.float32),
            pltpu.VMEM((RB, B), jnp.float32),
            pltpu.VMEM((RPW, K), jnp.float32),
            pltpu.VMEM((RPW, K), jnp.float32),
            pltpu.SemaphoreType.DMA,
            pltpu.SemaphoreType.DMA,
            pltpu.SemaphoreType.DMA,
            pltpu.SemaphoreType.DMA,
        ],
        compiler_params=_compiler_params(),
    )
    return run(f, x)


RBK = 256
CW = 128


def _tc_body(f_ref, x_ref, o_ref):
    xv = x_ref[...]
    x1 = jnp.floor(xv).astype(jnp.int32)
    dx = xv - x1.astype(jnp.float32)
    c1, l1 = lax.shift_right_logical(x1, 7), lax.bitwise_and(x1, 127)
    y1 = jnp.zeros((RBK, K), jnp.float32)
    for v in range(B // CW):
        fb = f_ref[:, v * CW:(v + 1) * CW]
        y1 = jnp.where(c1 == v, jnp.take_along_axis(fb, l1, axis=1), y1)
    o_ref[...] = y1

    @pl.when(jnp.any(dx != 0.0))
    def _slow():
        x2 = x1 + 1
        c2, l2 = lax.shift_right_logical(x2, 7), lax.bitwise_and(x2, 127)
        y2 = jnp.zeros((RBK, K), jnp.float32)
        for v in range(B // CW):
            fb = f_ref[:, v * CW:(v + 1) * CW]
            y2 = jnp.where(c2 == v, jnp.take_along_axis(fb, l2, axis=1), y2)
        o_ref[...] = y1 * (1.0 - dx) + y2 * dx


def _tc_half(f, x):
    grid = ((N - S) // RBK,)
    return pl.pallas_call(
        _tc_body,
        grid=grid,
        in_specs=[
            pl.BlockSpec((RBK, B), lambda i: (S // RBK + i, 0)),
            pl.BlockSpec((RBK, K), lambda i: (S // RBK + i, 0)),
        ],
        out_specs=pl.BlockSpec((RBK, K), lambda i: (i, 0)),
        out_shape=jax.ShapeDtypeStruct((N - S, K), jnp.float32),
    )(f, x)


@jax.jit
def kernel(f, x):
    return jnp.concatenate([_sc_half(f, x), _tc_half(f, x)], axis=0)
